# baseline (speedup 1.0000x reference)
import functools

import jax
import jax.numpy as jnp
from jax import lax
from jax.experimental import pallas as pl
from jax.experimental.pallas import tpu as pltpu
from jax.experimental.pallas import tpu_sc as plsc

NC, NS, LANES = 2, 16, 16
NW = NC * NS
CH = 128

_mesh = lambda: plsc.VectorSubcoreMesh(core_axis_name="c", subcore_axis_name="s",
                                       num_cores=NC, num_subcores=NS)


def _deg_call(dst_3d, zeros_1d, n_acc):
    nchunk = dst_3d.shape[1]

    def body(dst_hbm, z_hbm, out_hbm, ones_v, idx_d, acc_sh, sem0, sem1):
        cid = lax.axis_index("c")
        sid = lax.axis_index("s")
        wid = cid * NS + sid
        sems = (sem0, sem1)

        @pl.when(sid == 0)
        def _():
            pltpu.sync_copy(z_hbm, acc_sh)

        for j in range(CH // LANES):
            ones_v[pl.ds(j * LANES, LANES)] = jnp.ones((LANES,), jnp.float32)
        pltpu.sync_copy(dst_hbm.at[wid], idx_d)
        plsc.subcore_barrier()

        pltpu.async_copy(ones_v, acc_sh.at[idx_d.at[0]], sem0, add=True)

        def pair(j, carry):
            for b in (0, 1):
                i = 2 * j + b

                @pl.when(i + 1 < nchunk)
                def _():
                    pltpu.async_copy(ones_v, acc_sh.at[idx_d.at[i + 1]],
                                     sems[1 - b], add=True)

                pltpu.make_async_copy(ones_v, acc_sh.at[idx_d.at[i]],
                                      sems[b]).wait()
            return carry

        lax.fori_loop(0, nchunk // 2, pair, 0)
        plsc.subcore_barrier()

        @pl.when(sid == 0)
        def _():
            pltpu.sync_copy(acc_sh, out_hbm.at[pl.ds(cid * n_acc, n_acc)])

    f = pl.kernel(
        body,
        out_type=jax.ShapeDtypeStruct((NC * n_acc,), jnp.float32),
        mesh=_mesh(),
        scratch_types=[
            pltpu.VMEM((CH,), jnp.float32),
            pltpu.VMEM((nchunk, CH), jnp.int32),
            pltpu.VMEM_SHARED((n_acc,), jnp.float32),
            pltpu.SemaphoreType.DMA,
            pltpu.SemaphoreType.DMA,
        ],
    )
    return f(dst_3d, zeros_1d)


def _conv_call(g, zeros_2d, src_3d, dst_3d, n_acc):
    n, d = g.shape
    nchunk = src_3d.shape[1]

    def body(g_hbm, z_hbm, src_hbm, dst_hbm, out_hbm,
             idx_s, idx_d, rows0, rows1, acc_sh, sg0, sg1, ss0, ss1):
        cid = lax.axis_index("c")
        sid = lax.axis_index("s")
        wid = cid * NS + sid
        rows = (rows0, rows1)
        sg = (sg0, sg1)
        ss = (ss0, ss1)

        @pl.when((sid == 0) & (cid == 0))
        def _():
            pltpu.sync_copy(g_hbm, acc_sh.at[pl.ds(0, n)])

        @pl.when((sid == 0) & (cid != 0))
        def _():
            pltpu.sync_copy(z_hbm, acc_sh.at[pl.ds(0, n)])

        pltpu.sync_copy(src_hbm.at[wid], idx_s)
        pltpu.sync_copy(dst_hbm.at[wid], idx_d)
        plsc.subcore_barrier()

        pltpu.async_copy(g_hbm.at[idx_s.at[0]], rows0, sg0)

        def pair(j, carry):
            for b in (0, 1):
                i = 2 * j + b
                pltpu.make_async_copy(g_hbm.at[idx_s.at[i]], rows[b],
                                      sg[b]).wait()
                pltpu.async_copy(rows[b], acc_sh.at[idx_d.at[i]], ss[b],
                                 add=True)

                @pl.when(i >= 1)
                def _():
                    pltpu.make_async_copy(rows[1 - b],
                                          acc_sh.at[idx_d.at[i]],
                                          ss[1 - b]).wait()

                @pl.when(i + 1 < nchunk)
                def _():
                    pltpu.async_copy(g_hbm.at[idx_s.at[i + 1]], rows[1 - b],
                                     sg[1 - b])
            return carry

        lax.fori_loop(0, nchunk // 2, pair, 0)
        pltpu.make_async_copy(rows1, acc_sh.at[idx_d.at[0]], ss1).wait()
        plsc.subcore_barrier()

        @pl.when(sid == 0)
        def _():
            pltpu.sync_copy(acc_sh.at[pl.ds(0, n)], out_hbm.at[cid])

    f = pl.kernel(
        body,
        out_type=jax.ShapeDtypeStruct((NC, n, d), jnp.float32),
        mesh=_mesh(),
        compiler_params=pltpu.CompilerParams(use_tc_tiling_on_sc=False),
        scratch_types=[
            pltpu.VMEM((nchunk, CH), jnp.int32),
            pltpu.VMEM((nchunk, CH), jnp.int32),
            pltpu.VMEM((CH, d), jnp.float32),
            pltpu.VMEM((CH, d), jnp.float32),
            pltpu.VMEM_SHARED((n_acc, d), jnp.float32),
            pltpu.SemaphoreType.DMA,
            pltpu.SemaphoreType.DMA,
            pltpu.SemaphoreType.DMA,
            pltpu.SemaphoreType.DMA,
        ],
    )
    return f(g, zeros_2d, src_3d, dst_3d)


_BR = 1000


def _tc_first(deg_t, x, w1):
    n, in_dim = x.shape
    d = w1.shape[1]
    grid = (n // _BR,)

    def body(dp_ref, x_ref, w_ref, dis_ref, g_ref):
        dp = dp_ref[...]
        dis = lax.rsqrt(dp[:, :1] + dp[:, 1:] + 1.0)
        dis_ref[...] = dis
        g_ref[...] = dis * jnp.dot(x_ref[...], w_ref[...],
                                   preferred_element_type=jnp.float32)

    return pl.pallas_call(
        body,
        grid=grid,
        in_specs=[
            pl.BlockSpec((_BR, NC), lambda i: (i, 0)),
            pl.BlockSpec((_BR, in_dim), lambda i: (i, 0)),
            pl.BlockSpec((in_dim, d), lambda i: (0, 0)),
        ],
        out_specs=[
            pl.BlockSpec((_BR, 1), lambda i: (i, 0)),
            pl.BlockSpec((_BR, d), lambda i: (i, 0)),
        ],
        out_shape=[
            jax.ShapeDtypeStruct((n, 1), jnp.float32),
            jax.ShapeDtypeStruct((n, d), jnp.float32),
        ],
    )(deg_t, x, w1)


def _tc_mid(p, dis, b, w):
    _, n, d = p.shape
    d2 = w.shape[1]

    def body(p_ref, dis_ref, b_ref, w_ref, g_ref):
        dv = dis_ref[...]
        h = jnp.maximum(dv * (p_ref[0] + p_ref[1]) + b_ref[...], 0.0)
        g_ref[...] = dv * jnp.dot(h, w_ref[...],
                                  preferred_element_type=jnp.float32)

    return pl.pallas_call(
        body,
        grid=(n // _BR,),
        in_specs=[
            pl.BlockSpec((NC, _BR, d), lambda i: (0, i, 0)),
            pl.BlockSpec((_BR, 1), lambda i: (i, 0)),
            pl.BlockSpec((1, d), lambda i: (0, 0)),
            pl.BlockSpec((d, d2), lambda i: (0, 0)),
        ],
        out_specs=pl.BlockSpec((_BR, d2), lambda i: (i, 0)),
        out_shape=jax.ShapeDtypeStruct((n, d2), jnp.float32),
    )(p, dis, b, w)


def _tc_final(p, dis, b):
    _, n, d = p.shape

    def body(p_ref, dis_ref, b_ref, o_ref):
        o_ref[...] = dis_ref[...] * (p_ref[0] + p_ref[1]) + b_ref[...]

    return pl.pallas_call(
        body,
        grid=(n // _BR,),
        in_specs=[
            pl.BlockSpec((NC, _BR, d), lambda i: (0, i, 0)),
            pl.BlockSpec((_BR, 1), lambda i: (i, 0)),
            pl.BlockSpec((1, d), lambda i: (0, 0)),
        ],
        out_specs=pl.BlockSpec((_BR, d), lambda i: (i, 0)),
        out_shape=jax.ShapeDtypeStruct((n, d), jnp.float32),
    )(p, dis, b)


def kernel(x, edge_index, W1, b1, W2, b2, Wmu, bmu, Wlv, blv):
    n = x.shape[0]
    e = edge_index.shape[1]
    d = W1.shape[1]

    e_pad = -(-e // (NW * CH * 2)) * (NW * CH * 2)
    pad = e_pad - e
    nchunk = e_pad // (NW * CH)
    n_acc = n + LANES
    n_acc1 = -(-(n + 1) // 128) * 128

    src = edge_index[0].astype(jnp.int32)
    dst = edge_index[1].astype(jnp.int32)
    src_3d = jnp.concatenate([src, jnp.zeros((pad,), jnp.int32)]
                             ).reshape(NW, nchunk, CH)
    dst_3d = jnp.concatenate([dst, jnp.full((pad,), n, jnp.int32)]
                             ).reshape(NW, nchunk, CH)

    zeros_1d = jnp.zeros((n_acc1,), jnp.float32)
    zeros_2d = jnp.zeros((n, d), jnp.float32)

    deg_p = _deg_call(dst_3d, zeros_1d, n_acc1)
    dis, g1 = _tc_first(deg_p.reshape(NC, n_acc1)[:, :n].T, x, W1)

    p1 = _conv_call(g1, zeros_2d, src_3d, dst_3d, n_acc)
    g2 = _tc_mid(p1, dis, b1.reshape(1, d), W2)

    p2 = _conv_call(g2, zeros_2d, src_3d, dst_3d, n_acc)
    wc = jnp.concatenate([Wmu, Wlv], axis=1)
    g3 = _tc_mid(p2, dis, b2.reshape(1, d), wc)

    p3 = _conv_call(g3, zeros_2d, src_3d, dst_3d, n_acc)
    bc = jnp.concatenate([bmu, blv]).reshape(1, d)
    out = _tc_final(p3, dis, bc)

    z = Wmu.shape[1]
    return (out[:, :z], out[:, z:])
